# baseline (speedup 1.0000x reference)
import functools

import jax
import jax.numpy as jnp
from jax import lax
from jax.experimental import pallas as pl
from jax.experimental.pallas import tpu as pltpu
from jax.experimental.pallas import tpu_sc as plsc

VOCAB = 1_000_000
EMBED = 16
BATCH = 16384
SEQ = 200

NC = 2
NS = 16
NW = NC * NS
EP = BATCH // NW
E_IT = 16
TOK_IT = E_IT * SEQ
CHUNKS = (104, 96)
N_IT = EP // E_IT
UNROLL = 10


def _pool_body(tid_hbm, table_hbm, out_hbm, idx_v, rows_v, obuf_v, sem_g, sem_i):
    w = lax.axis_index("s") * NC + lax.axis_index("c")

    def issue_gathers(buf):
        for e in range(E_IT):
            o = 0
            for w_ in CHUNKS:
                pltpu.async_copy(
                    table_hbm.at[idx_v.at[buf, e, pl.ds(o, w_)]],
                    rows_v.at[buf, pl.ds(e * SEQ + o, w_)],
                    sem_g,
                )
                o += w_


    def drain_gathers(buf):
        pltpu.make_async_copy(
            table_hbm.at[pl.ds(0, TOK_IT)], rows_v.at[buf], sem_g
        ).wait()

    def start_idx_dma(blk, buf):
        pltpu.async_copy(tid_hbm.at[pl.ds(blk * E_IT, E_IT)], idx_v.at[buf], sem_i)

    def wait_idx_dma(buf):
        pltpu.make_async_copy(
            tid_hbm.at[pl.ds(0, E_IT)], idx_v.at[buf], sem_i
        ).wait()

    def accumulate(buf, blk):
        inv = jnp.float32(1.0 / SEQ)
        for e in range(E_IT):
            base = e * SEQ

            def body(t, ps):
                o = base + UNROLL * t
                return tuple(ps[k] + rows_v[buf, o + k] for k in range(UNROLL))

            ps = lax.fori_loop(
                0, SEQ // UNROLL, body,
                tuple(jnp.zeros((16,), jnp.float32) for _ in range(UNROLL)),
            )
            acc = (((ps[0] + ps[1]) + (ps[2] + ps[3]))
                   + ((ps[4] + ps[5]) + (ps[6] + ps[7]))) + (ps[8] + ps[9])
            obuf_v[e] = acc * inv
        pltpu.sync_copy(obuf_v, out_hbm.at[pl.ds(blk * E_IT, E_IT)])

    def step(i, buf):
        nbuf = 1 - buf
        blk = w * N_IT + i
        drain_gathers(buf)
        issue_gathers(nbuf)
        blk2 = jnp.minimum(blk + 2, (w + 1) * N_IT - 1)
        start_idx_dma(blk2, buf)
        accumulate(buf, blk)
        wait_idx_dma(buf)

    pltpu.sync_copy(tid_hbm.at[pl.ds(w * N_IT * E_IT, E_IT)], idx_v.at[0])
    issue_gathers(0)
    pltpu.sync_copy(tid_hbm.at[pl.ds((w * N_IT + 1) * E_IT, E_IT)], idx_v.at[1])

    def loop_body(k, _):
        step(2 * k, 0)
        step(2 * k + 1, 1)
        return _

    lax.fori_loop(0, N_IT // 2, loop_body, jnp.int32(0))
    drain_gathers(0)


@functools.partial(jax.jit, static_argnames=())
def _sc_pool(tid3, table):
    mesh = plsc.VectorSubcoreMesh(core_axis_name="c", subcore_axis_name="s")
    return pl.kernel(
        _pool_body,
        out_type=jax.ShapeDtypeStruct((BATCH, EMBED), jnp.float32),
        mesh=mesh,
        scratch_types=[
            pltpu.VMEM((2, E_IT, SEQ), jnp.int32),
            pltpu.VMEM((2, TOK_IT, EMBED), jnp.float32),
            pltpu.VMEM((E_IT, EMBED), jnp.float32),
            pltpu.SemaphoreType.DMA,
            pltpu.SemaphoreType.DMA,
        ],
        compiler_params=pltpu.CompilerParams(use_tc_tiling_on_sc=False),
    )(tid3, table)


def _mlp_body(x_ref, w1_ref, b1_ref, w2_ref, b2_ref, o_ref):
    x = x_ref[...]
    h = jnp.maximum(
        jnp.dot(x, w1_ref[...], preferred_element_type=jnp.float32)
        + b1_ref[...], 0.0)
    z = jnp.dot(h, w2_ref[...], preferred_element_type=jnp.float32) + b2_ref[...]
    o_ref[...] = jax.nn.sigmoid(z)


def _tc_mlp(x, W1, b1, W2, b2):
    BM = 4096
    grid = (BATCH // BM,)
    return pl.pallas_call(
        _mlp_body,
        grid=grid,
        in_specs=[
            pl.BlockSpec((BM, EMBED), lambda i: (i, 0)),
            pl.BlockSpec((EMBED, EMBED), lambda i: (0, 0)),
            pl.BlockSpec((1, EMBED), lambda i: (0, 0)),
            pl.BlockSpec((EMBED, 1), lambda i: (0, 0)),
            pl.BlockSpec((1, 1), lambda i: (0, 0)),
        ],
        out_specs=pl.BlockSpec((BM, 1), lambda i: (i, 0)),
        out_shape=jax.ShapeDtypeStruct((BATCH, 1), jnp.float32),
    )(x, W1, b1.reshape(1, EMBED), W2, b2.reshape(1, 1))


def kernel(token_ids, table, W1, b1, W2, b2):
    x = _sc_pool(token_ids.astype(jnp.int32), table)
    return _tc_mlp(x, W1, b1, W2, b2)


# speedup vs baseline: 1.0328x; 1.0328x over previous
import functools

import jax
import jax.numpy as jnp
from jax import lax
from jax.experimental import pallas as pl
from jax.experimental.pallas import tpu as pltpu
from jax.experimental.pallas import tpu_sc as plsc

VOCAB = 1_000_000
EMBED = 16
BATCH = 16384
SEQ = 200

NC = 2
NS = 16
NW = NC * NS
EP = BATCH // NW
E_IT = 16
TOK_IT = E_IT * SEQ
CHUNKS = (104, 96)
N_IT = EP // E_IT
UNROLL = 10


def _pool_body(tid_hbm, table_hbm, out_hbm, idx_v, rows_v, obuf_v,
               sem_g0, sem_g1, sem_i):
    w = lax.axis_index("s") * NC + lax.axis_index("c")
    sems = (sem_g0, sem_g1)

    def issue_gathers(buf):
        for e in range(E_IT):
            o = 0
            for w_ in CHUNKS:
                pltpu.async_copy(
                    table_hbm.at[idx_v.at[buf, e, pl.ds(o, w_)]],
                    rows_v.at[buf, pl.ds(e * SEQ + o, w_)],
                    sems[buf],
                )
                o += w_


    def drain_gathers(buf):
        pltpu.make_async_copy(
            table_hbm.at[pl.ds(0, TOK_IT)], rows_v.at[buf], sems[buf]
        ).wait()

    def start_idx_dma(blk, buf):
        pltpu.async_copy(tid_hbm.at[pl.ds(blk * E_IT, E_IT)], idx_v.at[buf], sem_i)

    def wait_idx_dma(buf):
        pltpu.make_async_copy(
            tid_hbm.at[pl.ds(0, E_IT)], idx_v.at[buf], sem_i
        ).wait()

    def accumulate(buf, blk):
        inv = jnp.float32(1.0 / SEQ)
        for e in range(E_IT):
            base = e * SEQ

            def body(t, ps):
                o = base + UNROLL * t
                return tuple(ps[k] + rows_v[buf, o + k] for k in range(UNROLL))

            ps = lax.fori_loop(
                0, SEQ // UNROLL, body,
                tuple(jnp.zeros((16,), jnp.float32) for _ in range(UNROLL)),
            )
            acc = (((ps[0] + ps[1]) + (ps[2] + ps[3]))
                   + ((ps[4] + ps[5]) + (ps[6] + ps[7]))) + (ps[8] + ps[9])
            obuf_v[e] = acc * inv
        pltpu.sync_copy(obuf_v, out_hbm.at[pl.ds(blk * E_IT, E_IT)])

    def step(i, buf):
        nbuf = 1 - buf
        blk = w * N_IT + i
        issue_gathers(nbuf)
        drain_gathers(buf)
        blk2 = jnp.minimum(blk + 2, (w + 1) * N_IT - 1)
        start_idx_dma(blk2, buf)
        accumulate(buf, blk)
        wait_idx_dma(buf)

    pltpu.sync_copy(tid_hbm.at[pl.ds(w * N_IT * E_IT, E_IT)], idx_v.at[0])
    issue_gathers(0)
    pltpu.sync_copy(tid_hbm.at[pl.ds((w * N_IT + 1) * E_IT, E_IT)], idx_v.at[1])

    def loop_body(k, _):
        step(2 * k, 0)
        step(2 * k + 1, 1)
        return _

    lax.fori_loop(0, N_IT // 2, loop_body, jnp.int32(0))
    drain_gathers(0)


@functools.partial(jax.jit, static_argnames=())
def _sc_pool(tid3, table):
    mesh = plsc.VectorSubcoreMesh(core_axis_name="c", subcore_axis_name="s")
    return pl.kernel(
        _pool_body,
        out_type=jax.ShapeDtypeStruct((BATCH, EMBED), jnp.float32),
        mesh=mesh,
        scratch_types=[
            pltpu.VMEM((2, E_IT, SEQ), jnp.int32),
            pltpu.VMEM((2, TOK_IT, EMBED), jnp.float32),
            pltpu.VMEM((E_IT, EMBED), jnp.float32),
            pltpu.SemaphoreType.DMA,
            pltpu.SemaphoreType.DMA,
            pltpu.SemaphoreType.DMA,
        ],
        compiler_params=pltpu.CompilerParams(use_tc_tiling_on_sc=False),
    )(tid3, table)


def _mlp_body(x_ref, w1_ref, b1_ref, w2_ref, b2_ref, o_ref):
    x = x_ref[...]
    h = jnp.maximum(
        jnp.dot(x, w1_ref[...], preferred_element_type=jnp.float32)
        + b1_ref[...], 0.0)
    z = jnp.dot(h, w2_ref[...], preferred_element_type=jnp.float32) + b2_ref[...]
    o_ref[...] = jax.nn.sigmoid(z)


def _tc_mlp(x, W1, b1, W2, b2):
    BM = 4096
    grid = (BATCH // BM,)
    return pl.pallas_call(
        _mlp_body,
        grid=grid,
        in_specs=[
            pl.BlockSpec((BM, EMBED), lambda i: (i, 0)),
            pl.BlockSpec((EMBED, EMBED), lambda i: (0, 0)),
            pl.BlockSpec((1, EMBED), lambda i: (0, 0)),
            pl.BlockSpec((EMBED, 1), lambda i: (0, 0)),
            pl.BlockSpec((1, 1), lambda i: (0, 0)),
        ],
        out_specs=pl.BlockSpec((BM, 1), lambda i: (i, 0)),
        out_shape=jax.ShapeDtypeStruct((BATCH, 1), jnp.float32),
    )(x, W1, b1.reshape(1, EMBED), W2, b2.reshape(1, 1))


def kernel(token_ids, table, W1, b1, W2, b2):
    x = _sc_pool(token_ids.astype(jnp.int32), table)
    return _tc_mlp(x, W1, b1, W2, b2)


# speedup vs baseline: 1.0331x; 1.0003x over previous
import functools

import jax
import jax.numpy as jnp
from jax import lax
from jax.experimental import pallas as pl
from jax.experimental.pallas import tpu as pltpu
from jax.experimental.pallas import tpu_sc as plsc

VOCAB = 1_000_000
EMBED = 16
BATCH = 16384
SEQ = 200

NC = 2
NS = 16
NW = NC * NS
EP = BATCH // NW
E_IT = 16
TOK_IT = E_IT * SEQ
CHUNKS = (104, 96)
N_IT = EP // E_IT
UNROLL = 10


def _pool_body(tid_hbm, table_hbm, out_hbm, idx_v, rows_v, obuf_v,
               sem_g0, sem_g1, sem_i):
    w = lax.axis_index("s") * NC + lax.axis_index("c")
    sems = (sem_g0, sem_g1)

    def issue_gathers(buf):
        for e in range(E_IT):
            o = 0
            for w_ in CHUNKS:
                pltpu.async_copy(
                    table_hbm.at[idx_v.at[buf, pl.ds(e * SEQ + o, w_)]],
                    rows_v.at[buf, pl.ds(e * SEQ + o, w_)],
                    sems[buf],
                )
                o += w_


    def drain_gathers(buf):
        pltpu.make_async_copy(
            table_hbm.at[pl.ds(0, TOK_IT)], rows_v.at[buf], sems[buf]
        ).wait()

    def start_idx_dma(blk, buf):
        pltpu.async_copy(
            tid_hbm.at[pl.ds(blk * TOK_IT, TOK_IT)], idx_v.at[buf], sem_i)

    def wait_idx_dma(buf):
        pltpu.make_async_copy(
            tid_hbm.at[pl.ds(0, TOK_IT)], idx_v.at[buf], sem_i
        ).wait()

    def accumulate(buf, blk):
        inv = jnp.float32(1.0 / SEQ)
        for e in range(E_IT):
            base = e * SEQ

            def body(t, ps):
                o = base + UNROLL * t
                return tuple(ps[k] + rows_v[buf, o + k] for k in range(UNROLL))

            ps = lax.fori_loop(
                0, SEQ // UNROLL, body,
                tuple(jnp.zeros((16,), jnp.float32) for _ in range(UNROLL)),
            )
            acc = (((ps[0] + ps[1]) + (ps[2] + ps[3]))
                   + ((ps[4] + ps[5]) + (ps[6] + ps[7]))) + (ps[8] + ps[9])
            obuf_v[e] = acc * inv
        pltpu.sync_copy(obuf_v, out_hbm.at[pl.ds(blk * E_IT, E_IT)])

    def step(i, buf):
        nbuf = 1 - buf
        blk = w * N_IT + i
        issue_gathers(nbuf)
        drain_gathers(buf)
        blk2 = jnp.minimum(blk + 2, (w + 1) * N_IT - 1)
        start_idx_dma(blk2, buf)
        accumulate(buf, blk)
        wait_idx_dma(buf)

    pltpu.sync_copy(tid_hbm.at[pl.ds(w * N_IT * TOK_IT, TOK_IT)], idx_v.at[0])
    issue_gathers(0)
    pltpu.sync_copy(
        tid_hbm.at[pl.ds((w * N_IT + 1) * TOK_IT, TOK_IT)], idx_v.at[1])

    def loop_body(k, _):
        step(2 * k, 0)
        step(2 * k + 1, 1)
        return _

    lax.fori_loop(0, N_IT // 2, loop_body, jnp.int32(0))
    drain_gathers(0)


@functools.partial(jax.jit, static_argnames=())
def _sc_pool(tid3, table):
    mesh = plsc.VectorSubcoreMesh(core_axis_name="c", subcore_axis_name="s")
    return pl.kernel(
        _pool_body,
        out_type=jax.ShapeDtypeStruct((BATCH, EMBED), jnp.float32),
        mesh=mesh,
        scratch_types=[
            pltpu.VMEM((2, TOK_IT), jnp.int32),
            pltpu.VMEM((2, TOK_IT, EMBED), jnp.float32),
            pltpu.VMEM((E_IT, EMBED), jnp.float32),
            pltpu.SemaphoreType.DMA,
            pltpu.SemaphoreType.DMA,
            pltpu.SemaphoreType.DMA,
        ],
        compiler_params=pltpu.CompilerParams(use_tc_tiling_on_sc=False),
    )(tid3, table)


def _mlp_body(x_ref, w1_ref, b1_ref, w2_ref, b2_ref, o_ref):
    x = x_ref[...]
    h = jnp.maximum(
        jnp.dot(x, w1_ref[...], preferred_element_type=jnp.float32)
        + b1_ref[...], 0.0)
    z = jnp.dot(h, w2_ref[...], preferred_element_type=jnp.float32) + b2_ref[...]
    o_ref[...] = jax.nn.sigmoid(z)


def _tc_mlp(x, W1, b1, W2, b2):
    BM = 4096
    grid = (BATCH // BM,)
    return pl.pallas_call(
        _mlp_body,
        grid=grid,
        in_specs=[
            pl.BlockSpec((BM, EMBED), lambda i: (i, 0)),
            pl.BlockSpec((EMBED, EMBED), lambda i: (0, 0)),
            pl.BlockSpec((1, EMBED), lambda i: (0, 0)),
            pl.BlockSpec((EMBED, 1), lambda i: (0, 0)),
            pl.BlockSpec((1, 1), lambda i: (0, 0)),
        ],
        out_specs=pl.BlockSpec((BM, 1), lambda i: (i, 0)),
        out_shape=jax.ShapeDtypeStruct((BATCH, 1), jnp.float32),
    )(x, W1, b1.reshape(1, EMBED), W2, b2.reshape(1, 1))


def kernel(token_ids, table, W1, b1, W2, b2):
    tid_flat = token_ids.astype(jnp.int32).reshape(BATCH * SEQ)
    x = _sc_pool(tid_flat, table)
    return _tc_mlp(x, W1, b1, W2, b2)
